# baseline (speedup 1.0000x reference)
import jax
import jax.numpy as jnp
from jax.experimental import pallas as pl
from jax.experimental.pallas import tpu as pltpu


def _hgnn_kernel(adj_ref, emb_ref, out_ref):
    h = pl.program_id(0)
    strip = adj_ref[...].astype(jnp.bfloat16)
    emb = emb_ref[...].astype(jnp.bfloat16)
    lat_blk = jax.lax.dot_general(
        strip, emb, (((0,), (0,)), ((), ())),
        preferred_element_type=jnp.float32)
    part = jax.lax.dot_general(
        strip, lat_blk.astype(jnp.bfloat16), (((1,), (0,)), ((), ())),
        preferred_element_type=jnp.float32)

    @pl.when(h == 0)
    def _init():
        out_ref[...] = part

    @pl.when(h != 0)
    def _acc():
        out_ref[...] += part


def kernel(adj, embeds):
    n, hh = adj.shape
    d = embeds.shape[1]
    bh = 512
    return pl.pallas_call(
        _hgnn_kernel,
        grid=(hh // bh,),
        in_specs=[
            pl.BlockSpec((n, bh), lambda h: (0, h)),
            pl.BlockSpec((n, d), lambda h: (0, 0)),
        ],
        out_specs=pl.BlockSpec((n, d), lambda h: (0, 0)),
        out_shape=jax.ShapeDtypeStruct((n, d), jnp.float32),
        compiler_params=pltpu.CompilerParams(
            vmem_limit_bytes=100 * 1024 * 1024),
    )(adj, embeds)


# speedup vs baseline: 1.4556x; 1.4556x over previous
import jax
import jax.numpy as jnp
from jax.experimental import pallas as pl
from jax.experimental.pallas import tpu as pltpu


def _hgnn_kernel(adj_ref, emb_ref, out_ref, emb16_ref, strip16_ref, lat16_ref):
    h = pl.program_id(0)
    nh = pl.num_programs(0)
    cur = h % 2
    prv = 1 - cur

    @pl.when(h == 0)
    def _cast_emb():
        emb16_ref[...] = emb_ref[...].astype(jnp.bfloat16)

    strip16_ref[cur] = adj_ref[...].astype(jnp.bfloat16)
    s16 = strip16_ref[cur]
    lat_blk = jax.lax.dot_general(
        s16, emb16_ref[...], (((0,), (0,)), ((), ())),
        preferred_element_type=jnp.float32)
    lat16_ref[cur] = lat_blk.astype(jnp.bfloat16)

    part = jax.lax.dot_general(
        strip16_ref[prv], lat16_ref[prv], (((1,), (0,)), ((), ())),
        preferred_element_type=jnp.float32)
    contrib = jnp.where(h > 0, part, 0.0)
    base = jnp.where(h > 1, out_ref[...], 0.0)
    out_ref[...] = base + contrib

    @pl.when(h == nh - 1)
    def _final():
        part2 = jax.lax.dot_general(
            s16, lat16_ref[cur], (((1,), (0,)), ((), ())),
            preferred_element_type=jnp.float32)
        out_ref[...] += part2


def kernel(adj, embeds):
    n, hh = adj.shape
    d = embeds.shape[1]
    bh = 256
    return pl.pallas_call(
        _hgnn_kernel,
        grid=(hh // bh,),
        in_specs=[
            pl.BlockSpec((n, bh), lambda h: (0, h)),
            pl.BlockSpec((n, d), lambda h: (0, 0)),
        ],
        out_specs=pl.BlockSpec((n, d), lambda h: (0, 0)),
        out_shape=jax.ShapeDtypeStruct((n, d), jnp.float32),
        scratch_shapes=[
            pltpu.VMEM((n, d), jnp.bfloat16),
            pltpu.VMEM((2, n, bh), jnp.bfloat16),
            pltpu.VMEM((2, bh, d), jnp.bfloat16),
        ],
    )(adj, embeds)
